# baseline (speedup 1.0000x reference)
import functools

import jax
import jax.numpy as jnp
from jax import lax
from jax.experimental import pallas as pl
from jax.experimental.pallas import tpu as pltpu
from jax.experimental.pallas import tpu_sc as plsc

EMBED_D = 128
CHUNK = 128
SLAB_CHUNKS = 2
SLAB = SLAB_CHUNKS * CHUNK
NBUF = 3
AHEAD = 2
NUM_CORES = 2
NUM_SUBCORES = 16
NW = NUM_CORES * NUM_SUBCORES


def _make_gather(n_rows: int, n_chunks: int):
  n_slabs = n_chunks // SLAB_CHUNKS
  mesh = plsc.VectorSubcoreMesh(core_axis_name="c", subcore_axis_name="s")

  @functools.partial(
      pl.kernel,
      out_type=jax.ShapeDtypeStruct((n_rows, EMBED_D), jnp.float32),
      mesh=mesh,
      scratch_types=[
          pltpu.VMEM((n_chunks, CHUNK), jnp.int32),
          pltpu.VMEM((NBUF, SLAB, EMBED_D), jnp.float32),
      ]
      + [pltpu.SemaphoreType.DMA] * (2 * NBUF),
  )
  def gather_kernel(table_hbm, idx_hbm, out_hbm, idx_v, rows_v, *sems):
    g_sems = sems[:NBUF]
    o_sems = sems[NBUF:]
    wid = lax.axis_index("s") * NUM_CORES + lax.axis_index("c")
    row_base = wid * (n_chunks * CHUNK)
    pltpu.sync_copy(idx_hbm.at[wid], idx_v)

    def start_slab(t, b):
      for i in range(SLAB_CHUNKS):
        pltpu.async_copy(table_hbm.at[idx_v.at[t * SLAB_CHUNKS + i]],
                         rows_v.at[b].at[pl.ds(i * CHUNK, CHUNK)], g_sems[b])

    def wait_slab(b):
      pltpu.make_async_copy(
          out_hbm.at[pl.ds(row_base, SLAB)], rows_v.at[b], g_sems[b]).wait()

    def start_out(t, b):
      pltpu.async_copy(rows_v.at[b],
                       out_hbm.at[pl.ds(row_base + t * SLAB, SLAB)],
                       o_sems[b])

    def wait_out(b):
      pltpu.make_async_copy(
          rows_v.at[b], out_hbm.at[pl.ds(row_base, SLAB)], o_sems[b]).wait()

    for t in range(AHEAD):
      start_slab(t, t)

    def step(t, b):
      wait_slab(b)
      start_out(t, b)
      nxt = t + AHEAD
      bn = (b + AHEAD) % NBUF

      @pl.when(jnp.logical_and(nxt < n_slabs, nxt >= NBUF))
      def _():
        wait_out(bn)

      @pl.when(nxt < n_slabs)
      def _():
        start_slab(nxt, bn)

    def superstep(g, carry):
      for k in range(NBUF):
        step(g * NBUF + k, k)
      return carry

    n_full = (n_slabs // NBUF) * NBUF
    lax.fori_loop(0, n_slabs // NBUF, superstep, None)
    for t in range(n_full, n_slabs):
      step(t, t % NBUF)

    for b in range(NBUF):
      wait_out(b)

  return gather_kernel


def kernel(codes, code_embed_weight):
  bsz, seq = codes.shape
  d = code_embed_weight.shape[1]
  idx = codes.T.astype(jnp.int32).reshape(-1)
  n_rows = idx.shape[0]
  n_chunks = n_rows // (NW * CHUNK)
  idx3 = idx.reshape(NW, n_chunks, CHUNK)
  out = _make_gather(n_rows, n_chunks)(code_embed_weight, idx3)
  return out.reshape(seq, bsz, d).transpose(1, 0, 2)


# speedup vs baseline: 1.0219x; 1.0219x over previous
import functools

import jax
import jax.numpy as jnp
from jax import lax
from jax.experimental import pallas as pl
from jax.experimental.pallas import tpu as pltpu
from jax.experimental.pallas import tpu_sc as plsc

EMBED_D = 128
COLS = 128
GCHUNK = 64
SPLIT = COLS // GCHUNK
NBUF = 10
AHEAD = 3
NUM_CORES = 2
NUM_SUBCORES = 16
NW = NUM_CORES * NUM_SUBCORES


def _make_gather(seq: int, n_batch: int):
  n_rows = seq * n_batch
  n_chunks = seq * SPLIT
  mesh = plsc.VectorSubcoreMesh(core_axis_name="c", subcore_axis_name="s")

  @functools.partial(
      pl.kernel,
      out_type=jax.ShapeDtypeStruct((n_rows, EMBED_D), jnp.float32),
      mesh=mesh,
      scratch_types=[
          pltpu.VMEM((seq, COLS), jnp.int32),
          pltpu.VMEM((NBUF, GCHUNK, EMBED_D), jnp.float32),
      ]
      + [pltpu.SemaphoreType.DMA] * (2 * NBUF),
  )
  def gather_kernel(table_hbm, idx_hbm, out_hbm, idx_v, rows_v, *sems):
    g_sems = sems[:NBUF]
    o_sems = sems[NBUF:]
    wid = lax.axis_index("s") * NUM_CORES + lax.axis_index("c")
    col_base = wid * COLS
    pltpu.sync_copy(idx_hbm.at[:, pl.ds(col_base, COLS)], idx_v)

    def idx_vec(j, h):
      return idx_v.at[j // SPLIT, pl.ds(h * GCHUNK, GCHUNK)]

    def start_gather(j, h, b):
      pltpu.async_copy(table_hbm.at[idx_vec(j, h)], rows_v.at[b], g_sems[b])

    def wait_gather(b):
      pltpu.make_async_copy(
          table_hbm.at[idx_vec(0, 0)], rows_v.at[b], g_sems[b]).wait()

    def start_out(j, h, b):
      pltpu.async_copy(
          rows_v.at[b],
          out_hbm.at[pl.ds((j // SPLIT) * n_batch + col_base + h * GCHUNK,
                           GCHUNK)],
          o_sems[b])

    def wait_out(b):
      pltpu.make_async_copy(
          rows_v.at[b], out_hbm.at[pl.ds(col_base, GCHUNK)], o_sems[b]).wait()

    for j in range(AHEAD):
      start_gather(j, j % SPLIT, j)

    def superstep(g, carry):
      for k in range(NBUF):
        j = g * NBUF + k
        h = k % SPLIT
        b = k
        wait_gather(b)
        start_out(j, h, b)
        nxt = j + AHEAD
        bn = (k + AHEAD) % NBUF

        @pl.when(jnp.logical_and(nxt < n_chunks, nxt >= NBUF))
        def _():
          wait_out(bn)

        @pl.when(nxt < n_chunks)
        def _():
          start_gather(nxt, (k + AHEAD) % SPLIT, bn)

      return carry

    lax.fori_loop(0, n_chunks // NBUF, superstep, None)

    for b in range(NBUF):
      wait_out(b)

  return gather_kernel


def kernel(codes, code_embed_weight):
  bsz, seq = codes.shape
  d = code_embed_weight.shape[1]
  codes_t = codes.T.astype(jnp.int32)
  out = _make_gather(seq, bsz)(code_embed_weight, codes_t)
  return out.reshape(seq, bsz, d).transpose(1, 0, 2)


# speedup vs baseline: 1.0327x; 1.0106x over previous
import functools

import jax
import jax.numpy as jnp
from jax import lax
from jax.experimental import pallas as pl
from jax.experimental.pallas import tpu as pltpu
from jax.experimental.pallas import tpu_sc as plsc

EMBED_D = 128
CHUNK = 128
NBUF = 5
AHEAD = 2
NUM_CORES = 2
NUM_SUBCORES = 16
NW = NUM_CORES * NUM_SUBCORES


def _make_gather(seq: int, n_batch: int):
  n_rows = seq * n_batch
  n_chunks = seq
  mesh = plsc.VectorSubcoreMesh(core_axis_name="c", subcore_axis_name="s")

  @functools.partial(
      pl.kernel,
      out_type=jax.ShapeDtypeStruct((n_rows, EMBED_D), jnp.float32),
      mesh=mesh,
      scratch_types=[
          pltpu.VMEM((n_chunks, CHUNK), jnp.int32),
          pltpu.VMEM((NBUF, CHUNK, EMBED_D), jnp.float32),
      ]
      + [pltpu.SemaphoreType.DMA] * (2 * NBUF),
  )
  def gather_kernel(table_hbm, idx_hbm, out_hbm, idx_v, rows_v, *sems):
    g_sems = sems[:NBUF]
    o_sems = sems[NBUF:]
    wid = lax.axis_index("s") * NUM_CORES + lax.axis_index("c")
    col_base = wid * CHUNK
    pltpu.sync_copy(idx_hbm.at[:, pl.ds(col_base, CHUNK)], idx_v)

    def start_gather(j, b):
      pltpu.async_copy(table_hbm.at[idx_v.at[j]], rows_v.at[b], g_sems[b])

    def wait_gather(b):
      pltpu.make_async_copy(
          table_hbm.at[idx_v.at[0]], rows_v.at[b], g_sems[b]).wait()

    def start_out(j, b):
      pltpu.async_copy(rows_v.at[b],
                       out_hbm.at[pl.ds(j * n_batch + col_base, CHUNK)],
                       o_sems[b])

    def wait_out(b):
      pltpu.make_async_copy(
          rows_v.at[b], out_hbm.at[pl.ds(col_base, CHUNK)], o_sems[b]).wait()

    for j in range(AHEAD):
      start_gather(j, j)

    def superstep(g, carry):
      for k in range(NBUF):
        j = g * NBUF + k
        b = k
        wait_gather(b)
        start_out(j, b)
        nxt = j + AHEAD
        bn = (k + AHEAD) % NBUF

        @pl.when(jnp.logical_and(nxt < n_chunks, nxt >= NBUF))
        def _():
          wait_out(bn)

        @pl.when(nxt < n_chunks)
        def _():
          start_gather(nxt, bn)

      return carry

    lax.fori_loop(0, n_chunks // NBUF, superstep, None)

    for b in range(NBUF):
      wait_out(b)

  return gather_kernel


def kernel(codes, code_embed_weight):
  bsz, seq = codes.shape
  d = code_embed_weight.shape[1]
  codes_t = codes.T.astype(jnp.int32)
  out = _make_gather(seq, bsz)(code_embed_weight, codes_t)
  return out.reshape(seq, bsz, d).transpose(1, 0, 2)
